# baseline (speedup 1.0000x reference)
import jax
import jax.numpy as jnp
from jax.experimental import pallas as pl
from jax.experimental.pallas import tpu as pltpu

D_MODEL = 1024
D_FF = 4096
N_EXP = 8
TOPK = 2
NBUF = 3


def _routing(token_ref, wr_ref, br_ref, dens_ref, u_ref):
    x = token_ref[...]
    logits = jnp.dot(x, wr_ref[...], preferred_element_type=jnp.float32)
    logits = logits + br_ref[...]
    m = jnp.max(logits)
    e = jnp.exp(logits - m)
    sm = e / jnp.sum(e)
    col8 = jax.lax.broadcasted_iota(jnp.int32, (1, N_EXP), 1)
    v0 = jnp.max(sm)
    first_max = jnp.min(jnp.where(sm == v0, col8, N_EXP))
    v1 = jnp.max(jnp.where(col8 == first_max, -jnp.inf, sm))
    flat = dens_ref[...]
    n = N_EXP * N_EXP
    r = jax.lax.broadcasted_iota(jnp.int32, (n, n), 0)
    c = jax.lax.broadcasted_iota(jnp.int32, (n, n), 1)
    tri = (r <= c).astype(jnp.float32)
    cum = jnp.dot(flat, tri, preferred_element_type=jnp.float32)
    col64 = jax.lax.broadcasted_iota(jnp.int32, (1, n), 1)
    c_last = jnp.sum(jnp.where(col64 == n - 1, cum, 0.0))
    u = u_ref[0, 0] * c_last
    idx = jnp.sum((cum < u).astype(jnp.int32))
    i0 = idx // N_EXP
    i1 = idx - N_EXP * i0
    return i0, i1, v0, v1


RB = 256
NW1 = D_MODEL // RB
CB = 1024
NW2 = D_FF // CB


def _moe_body(token_ref, wr_ref, br_ref, dens_ref, u_ref,
              w1_any, b1_any, w2_any, b2_any, out_ref,
              w1b, w2b, b1v, b2v, sem1, sem2, semb):
    i0, i1, v0, v1 = _routing(token_ref, wr_ref, br_ref, dens_ref, u_ref)
    eidx = [i0, i1]
    scales = [v0, v1]

    tasks = []
    for e in range(TOPK):
        tasks += [("w1", e, r) for r in range(NW1)]
        tasks += [("w2", e, j) for j in range(NW2)]

    slot_of = {}
    c1 = c2 = 0
    for s, (kind, e, i) in enumerate(tasks):
        if kind == "w1":
            slot_of[s] = c1 % NBUF
            c1 += 1
        else:
            slot_of[s] = c2 % NBUF
            c2 += 1

    def copy(s):
        kind, e, i = tasks[s]
        if kind == "w1":
            return pltpu.make_async_copy(
                w1_any.at[eidx[e], pl.ds(i * RB, RB), :],
                w1b.at[slot_of[s]], sem1.at[slot_of[s]])
        return pltpu.make_async_copy(
            w2_any.at[eidx[e], pl.ds(i * CB, CB), :],
            w2b.at[slot_of[s]], sem2.at[slot_of[s]])

    bias_copies = [
        pltpu.make_async_copy(b1_any.at[eidx[e]], b1v.at[e], semb)
        for e in range(TOPK)
    ] + [
        pltpu.make_async_copy(b2_any.at[eidx[e]], b2v.at[e], semb)
        for e in range(TOPK)
    ]
    for cp in bias_copies:
        cp.start()
    DEPTH = NBUF - 1
    for s in range(DEPTH):
        copy(s).start()
    for cp in bias_copies:
        cp.wait()

    x = token_ref[...]
    acc = v0 * b2v[0] + v1 * b2v[1]
    hparts = [None, None]
    hs = [None, None]
    for s, (kind, e, i) in enumerate(tasks):
        if s + DEPTH < len(tasks):
            copy(s + DEPTH).start()
        copy(s).wait()
        if kind == "w1":
            part = jnp.dot(x[:, i * RB:(i + 1) * RB], w1b[slot_of[s]],
                           preferred_element_type=jnp.float32)
            hparts[e] = part if hparts[e] is None else hparts[e] + part
            if i == NW1 - 1:
                hs[e] = jnp.maximum(hparts[e] + b1v[e], 0.0)
        else:
            acc = acc + scales[e] * jnp.dot(
                hs[e][:, i * CB:(i + 1) * CB], w2b[slot_of[s]],
                preferred_element_type=jnp.float32)
    out_ref[...] = acc


def _moe(tok2, Wr, br2, dflat, u, W1, b1, W2, b2):
    return pl.pallas_call(
        _moe_body,
        out_shape=jax.ShapeDtypeStruct((1, D_MODEL), jnp.float32),
        in_specs=[
            pl.BlockSpec(memory_space=pltpu.VMEM),
            pl.BlockSpec(memory_space=pltpu.VMEM),
            pl.BlockSpec(memory_space=pltpu.VMEM),
            pl.BlockSpec(memory_space=pltpu.VMEM),
            pl.BlockSpec(memory_space=pltpu.SMEM),
            pl.BlockSpec(memory_space=pl.ANY),
            pl.BlockSpec(memory_space=pl.ANY),
            pl.BlockSpec(memory_space=pl.ANY),
            pl.BlockSpec(memory_space=pl.ANY),
        ],
        out_specs=pl.BlockSpec(memory_space=pltpu.VMEM),
        scratch_shapes=[
            pltpu.VMEM((NBUF, RB, D_FF), jnp.float32),
            pltpu.VMEM((NBUF, CB, D_MODEL), jnp.float32),
            pltpu.VMEM((TOPK, 1, D_FF), jnp.float32),
            pltpu.VMEM((TOPK, 1, D_MODEL), jnp.float32),
            pltpu.SemaphoreType.DMA((NBUF,)),
            pltpu.SemaphoreType.DMA((NBUF,)),
            pltpu.SemaphoreType.DMA,
        ],
    )(tok2, Wr, br2, dflat, u, W1, b1.reshape(N_EXP, 1, D_FF), W2,
      b2.reshape(N_EXP, 1, D_MODEL))


def kernel(token, Wr, br, W1, b1, W2, b2, density):
    u = jax.random.uniform(jax.random.key(7), dtype=jnp.float32)
    u = u.reshape(1, 1)
    tok2 = token.reshape(1, D_MODEL)
    br2 = br.reshape(1, N_EXP)
    dflat = density.reshape(1, N_EXP * N_EXP)
    out = _moe(tok2, Wr, br2, dflat, u, W1, b1, W2, b2)
    return out.reshape(D_MODEL)
